# baseline (device time: 32915 ns/iter reference)
import jax
import jax.numpy as jnp
from jax import lax
from jax.experimental import pallas as pl
from jax.experimental.pallas import tpu as pltpu

_BF16 = jnp.bfloat16


def kernel(x):
    M, N = x.shape

    def body(
        x_ref,
        out_ref,
        xb,
        row_send,
        col_send,
        row_recv,
        col_recv,
        send_sems,
        recv_sems,
        ack_sem,
    ):
        sx = lax.axis_index("x")
        sy = lax.axis_index("y")

        @pl.when(sx == 0)
        def _():
            row_send[...] = x_ref[M - 1 : M, :].astype(_BF16)

        @pl.when(sx == 1)
        def _():
            row_send[...] = x_ref[0:1, :].astype(_BF16)

        @pl.when(sy == 0)
        def _():
            col_send[...] = x_ref[:, N - 1 : N].astype(_BF16)

        @pl.when(sy == 1)
        def _():
            col_send[...] = x_ref[:, 0:1].astype(_BF16)

        rdma_x = pltpu.make_async_remote_copy(
            src_ref=row_send,
            dst_ref=row_recv,
            send_sem=send_sems.at[0],
            recv_sem=recv_sems.at[0],
            device_id=(1 - sx, sy),
            device_id_type=pl.DeviceIdType.MESH,
        )
        rdma_y = pltpu.make_async_remote_copy(
            src_ref=col_send,
            dst_ref=col_recv,
            send_sem=send_sems.at[1],
            recv_sem=recv_sems.at[1],
            device_id=(sx, 1 - sy),
            device_id_type=pl.DeviceIdType.MESH,
        )
        rdma_x.start()
        rdma_y.start()

        xb[...] = x_ref[...].astype(_BF16)

        out_ref[1 : M - 1, :] = 0.5 * xb[1 : M - 1, :] + 0.125 * (
            xb[0 : M - 2, :] + xb[2:M, :]
        )
        out_ref[0:1, :] = 0.5 * xb[0:1, :] + 0.125 * xb[1:2, :]
        out_ref[M - 1 : M, :] = 0.5 * xb[M - 1 : M, :] + 0.125 * xb[M - 2 : M - 1, :]

        out_ref[:, 1 : N - 1] = out_ref[:, 1 : N - 1] + 0.125 * (
            xb[:, 0 : N - 2] + xb[:, 2:N]
        )
        out_ref[:, 0:1] = out_ref[:, 0:1] + 0.125 * xb[:, 1:2]
        out_ref[:, N - 1 : N] = out_ref[:, N - 1 : N] + 0.125 * xb[:, N - 2 : N - 1]

        rdma_x.wait()

        @pl.when(sx == 0)
        def _():
            out_ref[M - 1 : M, :] = out_ref[M - 1 : M, :] + 0.125 * row_recv[...]

        @pl.when(sx == 1)
        def _():
            out_ref[0:1, :] = out_ref[0:1, :] + 0.125 * row_recv[...]

        rdma_y.wait()

        @pl.when(sy == 0)
        def _():
            out_ref[:, N - 1 : N] = out_ref[:, N - 1 : N] + 0.125 * col_recv[...]

        @pl.when(sy == 1)
        def _():
            out_ref[:, 0:1] = out_ref[:, 0:1] + 0.125 * col_recv[...]

        @pl.when(sx == 0)
        def _():
            out_ref[0:1, :] = xb[0:1, :]

        @pl.when(sx == 1)
        def _():
            out_ref[M - 1 : M, :] = xb[M - 1 : M, :]

        @pl.when(sy == 0)
        def _():
            out_ref[:, 0:1] = xb[:, 0:1]

        @pl.when(sy == 1)
        def _():
            out_ref[:, N - 1 : N] = xb[:, N - 1 : N]

        pl.semaphore_signal(
            ack_sem,
            inc=1,
            device_id=(1 - sx, sy),
            device_id_type=pl.DeviceIdType.MESH,
        )
        pl.semaphore_signal(
            ack_sem,
            inc=1,
            device_id=(sx, 1 - sy),
            device_id_type=pl.DeviceIdType.MESH,
        )
        pl.semaphore_wait(ack_sem, 2)

    return pl.pallas_call(
        body,
        out_shape=jax.ShapeDtypeStruct((M, N), _BF16),
        in_specs=[pl.BlockSpec(memory_space=pltpu.VMEM)],
        out_specs=pl.BlockSpec(memory_space=pltpu.VMEM),
        scratch_shapes=[
            pltpu.VMEM((M, N), _BF16),
            pltpu.VMEM((1, N), _BF16),
            pltpu.VMEM((M, 1), _BF16),
            pltpu.VMEM((1, N), _BF16),
            pltpu.VMEM((M, 1), _BF16),
            pltpu.SemaphoreType.DMA((2,)),
            pltpu.SemaphoreType.DMA((2,)),
            pltpu.SemaphoreType.REGULAR,
        ],
        compiler_params=pltpu.CompilerParams(
            has_side_effects=True,
            vmem_limit_bytes=64 * 1024 * 1024,
        ),
    )(x)


# device time: 21850 ns/iter; 1.5064x vs baseline; 1.5064x over previous
import jax
import jax.numpy as jnp
from jax import lax
from jax.experimental import pallas as pl
from jax.experimental.pallas import tpu as pltpu

_BF16 = jnp.bfloat16

_R = 256
_B = 8


def _exchange(row_edge, col_edge):
    _, N = row_edge.shape
    _, M = col_edge.shape

    def body(
        row_send,
        col_send,
        rh_out,
        ch_out,
        row_recv,
        col_recv,
        send_sems,
        recv_sems,
        ack_sem,
    ):
        sx = lax.axis_index("x")
        sy = lax.axis_index("y")

        barrier_sem = pltpu.get_barrier_semaphore()
        pl.semaphore_signal(
            barrier_sem, inc=1, device_id=(1 - sx, sy),
            device_id_type=pl.DeviceIdType.MESH,
        )
        pl.semaphore_signal(
            barrier_sem, inc=1, device_id=(sx, 1 - sy),
            device_id_type=pl.DeviceIdType.MESH,
        )
        pl.semaphore_wait(barrier_sem, 2)

        rdma_x = pltpu.make_async_remote_copy(
            src_ref=row_send,
            dst_ref=row_recv,
            send_sem=send_sems.at[0],
            recv_sem=recv_sems.at[0],
            device_id=(1 - sx, sy),
            device_id_type=pl.DeviceIdType.MESH,
        )
        rdma_y = pltpu.make_async_remote_copy(
            src_ref=col_send,
            dst_ref=col_recv,
            send_sem=send_sems.at[1],
            recv_sem=recv_sems.at[1],
            device_id=(sx, 1 - sy),
            device_id_type=pl.DeviceIdType.MESH,
        )
        rdma_x.start()
        rdma_y.start()
        rdma_x.wait()
        rdma_y.wait()

        rh_out[...] = row_recv[...]
        ch_out[...] = col_recv[...]

        pl.semaphore_signal(
            ack_sem, inc=1, device_id=(1 - sx, sy),
            device_id_type=pl.DeviceIdType.MESH,
        )
        pl.semaphore_signal(
            ack_sem, inc=1, device_id=(sx, 1 - sy),
            device_id_type=pl.DeviceIdType.MESH,
        )
        pl.semaphore_wait(ack_sem, 2)

    return pl.pallas_call(
        body,
        out_shape=(
            jax.ShapeDtypeStruct((1, N), _BF16),
            jax.ShapeDtypeStruct((1, M), _BF16),
        ),
        in_specs=[
            pl.BlockSpec(memory_space=pltpu.VMEM),
            pl.BlockSpec(memory_space=pltpu.VMEM),
        ],
        out_specs=(
            pl.BlockSpec(memory_space=pltpu.VMEM),
            pl.BlockSpec(memory_space=pltpu.VMEM),
        ),
        scratch_shapes=[
            pltpu.VMEM((1, N), _BF16),
            pltpu.VMEM((1, M), _BF16),
            pltpu.SemaphoreType.DMA((2,)),
            pltpu.SemaphoreType.DMA((2,)),
            pltpu.SemaphoreType.REGULAR,
        ],
        compiler_params=pltpu.CompilerParams(collective_id=0),
    )(row_edge, col_edge)


def _stencil(x, row_halo, col_halo):
    M, N = x.shape
    assert _B * _R == M

    def window(b):
        if b == 0:
            return 0, _R + 8, 0
        return b * _R - 8, _R + 16 if b < _B - 1 else _R + 8, 8

    def body(
        x_hbm,
        rh,
        ch,
        out_hbm,
        xin,
        yb,
        chcol,
        in_sems,
        out_sems,
    ):
        sx = lax.axis_index("x")
        sy = lax.axis_index("y")

        chcol[...] = ch[...].reshape(M, 1)

        def start_in(b, slot):
            st, sz, _ = window(b)
            d = pltpu.make_async_copy(
                x_hbm.at[pl.ds(st, sz), :],
                xin.at[slot, pl.ds(0, sz), :],
                in_sems.at[slot],
            )
            d.start()
            return d

        in_d = {0: start_in(0, 0), 1: start_in(1, 1)}
        out_d = {}

        lane = lax.broadcasted_iota(jnp.int32, (1, N), 1)

        def line(xt, other, halo_corner):
            rhv = rh[...]
            w = jnp.concatenate([xt[:, 0:1], xt[:, 0 : N - 1]], axis=1)
            e = jnp.concatenate([xt[:, 1:N], xt[:, N - 1 : N]], axis=1)
            st = 0.5 * xt + 0.125 * (other + rhv + w + e)
            c0 = jnp.where(
                sy == 0,
                xt[:, 0:1],
                0.5 * xt[:, 0:1]
                + 0.125
                * (other[:, 0:1] + rhv[:, 0:1] + halo_corner + xt[:, 1:2]),
            )
            cN = jnp.where(
                sy == 1,
                xt[:, N - 1 : N],
                0.5 * xt[:, N - 1 : N]
                + 0.125
                * (
                    other[:, N - 1 : N]
                    + rhv[:, N - 1 : N]
                    + xt[:, N - 2 : N - 1]
                    + halo_corner
                ),
            )
            return jnp.where(lane == 0, c0, jnp.where(lane == N - 1, cN, st))

        for b in range(_B):
            slot = b % 3
            if b + 2 < _B:
                in_d[b + 2] = start_in(b + 2, (b + 2) % 3)
            in_d[b].wait()
            oslot = b % 2
            if b >= 2:
                out_d[b - 2].wait()

            st_, sz, lo = window(b)
            k0 = 1 if b == 0 else 0
            k1 = _R - 1 if b == _B - 1 else _R
            cnt = k1 - k0
            base = lo + k0

            xb = xin[slot, 0:sz, :].astype(_BF16)
            n = xb[base - 1 : base - 1 + cnt, :]
            c = xb[base : base + cnt, :]
            s = xb[base + 1 : base + 1 + cnt, :]
            core = 0.5 * c + 0.125 * (n + s)
            yb[oslot, k0:k1, 1 : N - 1] = core[:, 1 : N - 1] + 0.125 * (
                c[:, 0 : N - 2] + c[:, 2:N]
            )
            chs = chcol[pl.ds(b * _R + k0, cnt), 0:1]

            @pl.when(sy == 0)
            def _():
                yb[oslot, k0:k1, 0:1] = c[:, 0:1]
                yb[oslot, k0:k1, N - 1 : N] = 0.5 * c[:, N - 1 : N] + 0.125 * (
                    n[:, N - 1 : N]
                    + s[:, N - 1 : N]
                    + c[:, N - 2 : N - 1]
                    + chs
                )

            @pl.when(sy == 1)
            def _():
                yb[oslot, k0:k1, 0:1] = 0.5 * c[:, 0:1] + 0.125 * (
                    n[:, 0:1] + s[:, 0:1] + c[:, 1:2] + chs
                )
                yb[oslot, k0:k1, N - 1 : N] = c[:, N - 1 : N]

            if b == 0:
                ln = line(xb[0:1, :], xb[1:2, :], ch[0:1, 0:1])
                yb[oslot, 0:1, :] = jnp.where(sx == 0, xb[0:1, :], ln)
            if b == _B - 1:
                ln = line(
                    xb[sz - 1 : sz, :], xb[sz - 2 : sz - 1, :],
                    ch[0:1, M - 1 : M],
                )
                yb[oslot, _R - 1 : _R, :] = jnp.where(
                    sx == 1, xb[sz - 1 : sz, :], ln
                )

            out_d[b] = pltpu.make_async_copy(
                yb.at[oslot],
                out_hbm.at[pl.ds(b * _R, _R), :],
                out_sems.at[oslot],
            )
            out_d[b].start()

        out_d[_B - 2].wait()
        out_d[_B - 1].wait()

    return pl.pallas_call(
        body,
        out_shape=jax.ShapeDtypeStruct((M, N), _BF16),
        in_specs=[
            pl.BlockSpec(memory_space=pltpu.MemorySpace.HBM),
            pl.BlockSpec(memory_space=pltpu.VMEM),
            pl.BlockSpec(memory_space=pltpu.VMEM),
        ],
        out_specs=pl.BlockSpec(memory_space=pltpu.MemorySpace.HBM),
        scratch_shapes=[
            pltpu.VMEM((3, _R + 16, N), jnp.float32),
            pltpu.VMEM((2, _R, N), _BF16),
            pltpu.VMEM((M, 1), _BF16),
            pltpu.SemaphoreType.DMA((3,)),
            pltpu.SemaphoreType.DMA((2,)),
        ],
        compiler_params=pltpu.CompilerParams(
            vmem_limit_bytes=64 * 1024 * 1024
        ),
    )(x, row_halo, col_halo)


def kernel(x):
    M, N = x.shape
    sx = lax.axis_index("x")
    sy = lax.axis_index("y")
    ridx = jnp.where(sx == 0, M - 1, 0)
    row_edge = lax.dynamic_slice(x, (ridx, 0), (1, N)).astype(_BF16)
    cidx = jnp.where(sy == 0, N - 1, 0)
    col_edge = (
        lax.dynamic_slice(x, (0, cidx), (M, 1)).reshape(1, M).astype(_BF16)
    )
    row_halo, col_halo = _exchange(row_edge, col_edge)
    return _stencil(x, row_halo, col_halo)
